# baseline (speedup 1.0000x reference)
import functools

import jax
import jax.numpy as jnp
from jax import lax
from jax.experimental import pallas as pl
from jax.experimental.pallas import tpu as pltpu
from jax.experimental.pallas import tpu_sc as plsc

NC = 2
NS = 16
NW = NC * NS


def _transform_body(x_ref, w_ref, t_ref):
    t_ref[0] = jnp.dot(x_ref[...], w_ref[0],
                       preferred_element_type=jnp.float32)


def _transform(x, w_all, n_blk):
    n, d = x.shape
    k, _, do = w_all.shape
    grid = n // n_blk
    return pl.pallas_call(
        _transform_body,
        grid=(grid, k),
        in_specs=[
            pl.BlockSpec((n_blk, d), lambda i, j: (i, 0)),
            pl.BlockSpec((1, d, do), lambda i, j: (j, 0, 0)),
        ],
        out_specs=pl.BlockSpec((1, n_blk, do), lambda i, j: (j, i, 0)),
        out_shape=jax.ShapeDtypeStruct((k, n, do), jnp.float32),
    )(x, w_all)


def _make_sc_agg(n_nodes, n_edges, do, r):
    MC = 2000
    GC = 80
    NSUB = MC // GC
    epw = n_edges // NW
    nmeta = epw // MC
    ZR = 80
    RPT = 640
    assert epw % MC == 0 and MC % GC == 0 and GC % 16 == 0
    assert (NS - 1) * RPT < n_nodes <= NS * RPT
    assert (n_nodes - (NS - 1) * RPT) % ZR == 0 and RPT % ZR == 0

    mesh = plsc.VectorSubcoreMesh(core_axis_name="c", subcore_axis_name="s")

    @functools.partial(
        pl.kernel,
        out_type=jax.ShapeDtypeStruct((NC, n_nodes, do), jnp.float32),
        mesh=mesh,
        scratch_types=[
            pltpu.VMEM((MC,), jnp.int32),
            pltpu.VMEM((MC,), jnp.int32),
            pltpu.VMEM((MC,), jnp.int32),
            pltpu.VMEM((NSUB, GC), jnp.int32),
            pltpu.VMEM((GC,), jnp.int32),
            pltpu.VMEM((GC,), jnp.int32),
            pltpu.VMEM((GC, do), jnp.float32),
            pltpu.VMEM((GC, do), jnp.float32),
            pltpu.VMEM((ZR, do), jnp.float32),
            pltpu.VMEM_SHARED((n_nodes, do), jnp.float32),
            pltpu.SemaphoreType.DMA,
            pltpu.SemaphoreType.DMA,
        ],
    )
    def sc_agg(t_hbm, src_hbm, dstm_hbm, et_hbm, out_hbm,
               src_v, dstm_v, et_v, dst_v, gidx0_v, gidx1_v, rows0_v,
               rows1_v, zbuf_v, acc_sh, sem0, sem1):
        cid = lax.axis_index("c")
        sid = lax.axis_index("s")
        wid = sid * NC + cid

        zero16 = jnp.zeros((16,), jnp.float32)

        def zrow(i, carry):
            for j in range(do // 16):
                zbuf_v[i, pl.ds(j * 16, 16)] = zero16
            return carry

        lax.fori_loop(0, ZR, zrow, 0)
        row0 = sid * RPT
        ncopies = jnp.where(sid < NS - 1, RPT // ZR,
                            (n_nodes - (NS - 1) * RPT) // ZR)

        def zcopy(k, carry):
            pltpu.sync_copy(
                zbuf_v, acc_sh.at[pl.ds(pl.multiple_of(row0 + k * ZR, ZR), ZR)])
            return carry

        lax.fori_loop(0, ncopies, zcopy, 0)
        plsc.subcore_barrier()

        ebase = wid * epw
        nchunks = epw // GC

        def load_meta(mr):
            base = pl.multiple_of(ebase + mr * MC, MC)
            pltpu.sync_copy(src_hbm.at[pl.ds(base, MC)], src_v)
            pltpu.sync_copy(et_hbm.at[pl.ds(base, MC)], et_v)
            pltpu.sync_copy(dstm_hbm.at[pl.ds(base, MC)], dstm_v)

        def prep_fire(c, gidx_b, rows_b, sem_b):
            @pl.when(c % NSUB == 0)
            def _():
                load_meta(c // NSUB)

            rr = c % NSUB
            goff = rr * GC
            for j in range(GC // 16):
                s = src_v[pl.ds(goff + j * 16, 16)]
                t = et_v[pl.ds(goff + j * 16, 16)]
                gidx_b[pl.ds(j * 16, 16)] = t * n_nodes + s
                dst_v[rr, pl.ds(j * 16, 16)] = dstm_v[pl.ds(goff + j * 16, 16)]
            pltpu.async_copy(t_hbm.at[gidx_b], rows_b, sem_b)

        def drain(c, gidx_b, rows_b, sem_b):
            pltpu.make_async_copy(t_hbm.at[gidx_b], rows_b, sem_b).wait()
            pltpu.sync_copy(rows_b, acc_sh.at[dst_v.at[c % NSUB]], add=True)

        prep_fire(0, gidx0_v, rows0_v, sem0)

        def pair(h, carry):
            c = 2 * h
            prep_fire(c + 1, gidx1_v, rows1_v, sem1)
            drain(c, gidx0_v, rows0_v, sem0)
            prep_fire(c + 2, gidx0_v, rows0_v, sem0)
            drain(c + 1, gidx1_v, rows1_v, sem1)
            return carry

        lax.fori_loop(0, (nchunks - 1) // 2, pair, 0)
        drain(nchunks - 1, gidx0_v, rows0_v, sem0)
        plsc.subcore_barrier()

        def wcopy(k, carry):
            off = pl.multiple_of(row0 + k * ZR, ZR)
            pltpu.sync_copy(acc_sh.at[pl.ds(off, ZR)],
                            out_hbm.at[cid, pl.ds(off, ZR)])
            return carry

        lax.fori_loop(0, ncopies, wcopy, 0)

    return sc_agg


def _combine_body(p_ref, lp_ref, b_ref, o_ref):
    p = p_ref[...]
    o_ref[...] = p[0] + p[1] + lp_ref[0] + b_ref[...]


def _combine(partials, t_all, b, n_blk):
    k, n, do = t_all.shape
    grid = n // n_blk
    return pl.pallas_call(
        _combine_body,
        grid=(grid,),
        in_specs=[
            pl.BlockSpec((NC, n_blk, do), lambda i: (0, i, 0)),
            pl.BlockSpec((1, n_blk, do), lambda i: (k - 1, i, 0)),
            pl.BlockSpec((1, do), lambda i: (0, 0)),
        ],
        out_specs=pl.BlockSpec((n_blk, do), lambda i: (i, 0)),
        out_shape=jax.ShapeDtypeStruct((n, do), jnp.float32),
    )(partials, t_all, b.reshape(1, do))


def kernel(x, edge_index, etypes, W, W_loop, b):
    n, d = x.shape
    r, _, do = W.shape
    e = etypes.shape[0]

    w_all = jnp.concatenate([W, W_loop[None]], axis=0)

    t_all = _transform(x, w_all, n_blk=1000)
    t_rows = t_all.reshape((r + 1) * n, do)

    src = edge_index[0].astype(jnp.int32)
    dst = edge_index[1].astype(jnp.int32)
    et = etypes.astype(jnp.int32)

    partials = _make_sc_agg(n, e, do, r)(t_rows, src, dst, et)
    return _combine(partials, t_all, b, n_blk=1000)


# speedup vs baseline: 1.0721x; 1.0721x over previous
import functools

import jax
import jax.numpy as jnp
from jax import lax
from jax.experimental import pallas as pl
from jax.experimental.pallas import tpu as pltpu
from jax.experimental.pallas import tpu_sc as plsc

NC = 2
NS = 16
NW = NC * NS


def _transform_body(x_ref, w_ref, t_ref):
    t_ref[0] = jnp.dot(x_ref[...], w_ref[0],
                       preferred_element_type=jnp.float32)


def _transform(x, w_all, n_blk):
    n, d = x.shape
    k, _, do = w_all.shape
    grid = n // n_blk
    return pl.pallas_call(
        _transform_body,
        grid=(grid, k),
        in_specs=[
            pl.BlockSpec((n_blk, d), lambda i, j: (i, 0)),
            pl.BlockSpec((1, d, do), lambda i, j: (j, 0, 0)),
        ],
        out_specs=pl.BlockSpec((1, n_blk, do), lambda i, j: (j, i, 0)),
        out_shape=jax.ShapeDtypeStruct((k, n, do), jnp.float32),
    )(x, w_all)


def _make_sc_agg(n_nodes, n_edges, do, r):
    MC = 2000
    GC = 80
    NSUB = MC // GC
    epw = n_edges // NW
    nmeta = epw // MC
    ZR = 40
    RPT = 640
    assert epw % MC == 0 and MC % GC == 0 and GC % 16 == 0
    assert (NS - 1) * RPT < n_nodes <= NS * RPT
    assert (n_nodes - (NS - 1) * RPT) % ZR == 0 and RPT % ZR == 0

    mesh = plsc.VectorSubcoreMesh(core_axis_name="c", subcore_axis_name="s")

    @functools.partial(
        pl.kernel,
        out_type=jax.ShapeDtypeStruct((NC, n_nodes, do), jnp.float32),
        mesh=mesh,
        scratch_types=[
            pltpu.VMEM((MC,), jnp.int32),
            pltpu.VMEM((MC,), jnp.int32),
            pltpu.VMEM((MC,), jnp.int32),
            pltpu.VMEM((NSUB, GC), jnp.int32),
            pltpu.VMEM((GC,), jnp.int32),
            pltpu.VMEM((GC,), jnp.int32),
            pltpu.VMEM((GC,), jnp.int32),
            pltpu.VMEM((GC, do), jnp.float32),
            pltpu.VMEM((GC, do), jnp.float32),
            pltpu.VMEM((GC, do), jnp.float32),
            pltpu.VMEM((ZR, do), jnp.float32),
            pltpu.VMEM_SHARED((n_nodes, do), jnp.float32),
            pltpu.SemaphoreType.DMA,
            pltpu.SemaphoreType.DMA,
            pltpu.SemaphoreType.DMA,
        ],
    )
    def sc_agg(t_hbm, src_hbm, dstm_hbm, et_hbm, out_hbm,
               src_v, dstm_v, et_v, dst_v, gidx0_v, gidx1_v, gidx2_v,
               rows0_v, rows1_v, rows2_v, zbuf_v, acc_sh, sem0, sem1,
               sem2):
        cid = lax.axis_index("c")
        sid = lax.axis_index("s")
        wid = sid * NC + cid

        zero16 = jnp.zeros((16,), jnp.float32)

        def zrow(i, carry):
            for j in range(do // 16):
                zbuf_v[i, pl.ds(j * 16, 16)] = zero16
            return carry

        lax.fori_loop(0, ZR, zrow, 0)
        row0 = sid * RPT
        ncopies = jnp.where(sid < NS - 1, RPT // ZR,
                            (n_nodes - (NS - 1) * RPT) // ZR)

        def zcopy(k, carry):
            pltpu.sync_copy(
                zbuf_v, acc_sh.at[pl.ds(pl.multiple_of(row0 + k * ZR, ZR), ZR)])
            return carry

        lax.fori_loop(0, ncopies, zcopy, 0)
        plsc.subcore_barrier()

        ebase = wid * epw
        nchunks = epw // GC

        def load_meta(mr):
            base = pl.multiple_of(ebase + mr * MC, MC)
            pltpu.sync_copy(src_hbm.at[pl.ds(base, MC)], src_v)
            pltpu.sync_copy(et_hbm.at[pl.ds(base, MC)], et_v)
            pltpu.sync_copy(dstm_hbm.at[pl.ds(base, MC)], dstm_v)

        def prep_fire(c, gidx_b, rows_b, sem_b):
            @pl.when(c % NSUB == 0)
            def _():
                load_meta(c // NSUB)

            rr = c % NSUB
            goff = rr * GC
            for j in range(GC // 16):
                s = src_v[pl.ds(goff + j * 16, 16)]
                t = et_v[pl.ds(goff + j * 16, 16)]
                gidx_b[pl.ds(j * 16, 16)] = t * n_nodes + s
                dst_v[rr, pl.ds(j * 16, 16)] = dstm_v[pl.ds(goff + j * 16, 16)]
            pltpu.async_copy(t_hbm.at[gidx_b], rows_b, sem_b)

        def drain(c, gidx_b, rows_b, sem_b):
            pltpu.make_async_copy(t_hbm.at[gidx_b], rows_b, sem_b).wait()
            pltpu.sync_copy(rows_b, acc_sh.at[dst_v.at[c % NSUB]], add=True)

        bufs = [(gidx0_v, rows0_v, sem0), (gidx1_v, rows1_v, sem1),
                (gidx2_v, rows2_v, sem2)]
        prep_fire(0, *bufs[0])
        prep_fire(1, *bufs[1])

        def triple(h, carry):
            c0 = 3 * h
            for u in range(3):
                prep_fire(c0 + u + 2, *bufs[(u + 2) % 3])
                drain(c0 + u, *bufs[u])
            return carry

        nt = (nchunks - 2) // 3
        assert 3 * nt + 2 == nchunks
        lax.fori_loop(0, nt, triple, 0)
        drain(nchunks - 2, *bufs[(nchunks - 2) % 3])
        drain(nchunks - 1, *bufs[(nchunks - 1) % 3])
        plsc.subcore_barrier()

        def wcopy(k, carry):
            off = pl.multiple_of(row0 + k * ZR, ZR)
            pltpu.sync_copy(acc_sh.at[pl.ds(off, ZR)],
                            out_hbm.at[cid, pl.ds(off, ZR)])
            return carry

        lax.fori_loop(0, ncopies, wcopy, 0)

    return sc_agg


def _combine_body(p_ref, lp_ref, b_ref, o_ref):
    p = p_ref[...]
    o_ref[...] = p[0] + p[1] + lp_ref[0] + b_ref[...]


def _combine(partials, t_all, b, n_blk):
    k, n, do = t_all.shape
    grid = n // n_blk
    return pl.pallas_call(
        _combine_body,
        grid=(grid,),
        in_specs=[
            pl.BlockSpec((NC, n_blk, do), lambda i: (0, i, 0)),
            pl.BlockSpec((1, n_blk, do), lambda i: (k - 1, i, 0)),
            pl.BlockSpec((1, do), lambda i: (0, 0)),
        ],
        out_specs=pl.BlockSpec((n_blk, do), lambda i: (i, 0)),
        out_shape=jax.ShapeDtypeStruct((n, do), jnp.float32),
    )(partials, t_all, b.reshape(1, do))


def kernel(x, edge_index, etypes, W, W_loop, b):
    n, d = x.shape
    r, _, do = W.shape
    e = etypes.shape[0]

    w_all = jnp.concatenate([W, W_loop[None]], axis=0)

    t_all = _transform(x, w_all, n_blk=1000)
    t_rows = t_all.reshape((r + 1) * n, do)

    src = edge_index[0].astype(jnp.int32)
    dst = edge_index[1].astype(jnp.int32)
    et = etypes.astype(jnp.int32)

    partials = _make_sc_agg(n, e, do, r)(t_rows, src, dst, et)
    return _combine(partials, t_all, b, n_blk=1000)
